# baseline (speedup 1.0000x reference)
import functools

import numpy as np

import jax
import jax.numpy as jnp
from jax import lax
from jax.experimental import pallas as pl
from jax.experimental.pallas import tpu as pltpu
from jax.experimental.pallas import tpu_sc as plsc

N = 10000
E = 320000
IN_DIM = 128
OUT = 64
K = 12

NC = 2
NS = 16
NW = NC * NS
NPAD = 10240
RPS = NPAD // NS
EPW = E // NW
C = 80
NCHUNK = EPW // C
FPT = OUT // NW
CH = 6400
NCHK = E // CH

def _deg_body(dst_hbm, zeros_hbm, ones_hbm, out_hbm, valb, dstb, acc_sh):
    c = lax.axis_index("c")
    s = lax.axis_index("s")
    w = c * jnp.int32(NS) + s
    pltpu.sync_copy(zeros_hbm, valb)
    for j in range(RPS // C):
        pltpu.sync_copy(valb, acc_sh.at[pl.ds(s * jnp.int32(RPS) + jnp.int32(j * C), C)])
    plsc.subcore_barrier()
    pltpu.sync_copy(ones_hbm, valb)

    @pl.loop(jnp.int32(0), jnp.int32(NCHUNK))
    def _chunks(i):
        base = w * jnp.int32(EPW) + i * jnp.int32(C)
        pltpu.sync_copy(dst_hbm.at[pl.ds(base, C)], dstb)
        pltpu.sync_copy(valb, acc_sh.at[dstb], add=True)
    plsc.subcore_barrier()
    for j in range(RPS // C):
        pltpu.sync_copy(acc_sh.at[pl.ds(s * jnp.int32(RPS) + jnp.int32(j * C), C)], valb)
        pltpu.sync_copy(valb, out_hbm.at[pl.ds(c * jnp.int32(NPAD) + s * jnp.int32(RPS) + jnp.int32(j * C), C)])


def _hop_body(xt_hbm, src_hbm, dst_hbm, zeros2_hbm, out_hbm,
              xsl, acc, sb0, db0, sb1, db1, i0, i1, i2, i3):
    c = lax.axis_index("c")
    s = lax.axis_index("s")
    t2 = (c * jnp.int32(NS) + s) * jnp.int32(FPT)
    pltpu.sync_copy(xt_hbm.at[pl.ds(t2, FPT)], xsl)
    pltpu.sync_copy(zeros2_hbm, acc)

    F0 = jnp.full((16,), 0, jnp.int32)
    F1 = jnp.full((16,), 1, jnp.int32)

    def issue(ch, sbuf, dbuf, ss, sd):
        base = ch * jnp.int32(CH)
        pltpu.async_copy(src_hbm.at[pl.ds(base, CH)], sbuf, ss)
        pltpu.async_copy(dst_hbm.at[pl.ds(base, CH)], dbuf, sd)

    def drain(sbuf, dbuf, ss, sd):
        pltpu.make_async_copy(
            src_hbm.at[pl.ds(jnp.int32(0), CH)], sbuf, ss).wait()
        pltpu.make_async_copy(
            dst_hbm.at[pl.ds(jnp.int32(0), CH)], dbuf, sd).wait()

    def compute(sbuf, dbuf):
        @pl.loop(jnp.int32(0), jnp.int32(CH // 16))
        def _vecs(k):
            off = k * jnp.int32(16)
            sv = sbuf[pl.ds(off, 16)]
            dv = dbuf[pl.ds(off, 16)]
            v0 = plsc.load_gather(xsl, [F0, sv])
            plsc.addupdate_scatter(acc, [F0, dv], v0)
            v1 = plsc.load_gather(xsl, [F1, sv])
            plsc.addupdate_scatter(acc, [F1, dv], v1)

    issue(jnp.int32(0), sb0, db0, i0, i1)

    @pl.loop(jnp.int32(0), jnp.int32(NCHK // 2))
    def _pairs(i):
        ch = i * jnp.int32(2)
        issue(ch + jnp.int32(1), sb1, db1, i2, i3)
        drain(sb0, db0, i0, i1)
        compute(sb0, db0)
        issue(jnp.minimum(ch + jnp.int32(2), jnp.int32(NCHK - 2)),
              sb0, db0, i0, i1)
        drain(sb1, db1, i2, i3)
        compute(sb1, db1)

    drain(sb0, db0, i0, i1)
    pltpu.sync_copy(acc, out_hbm.at[pl.ds(t2, FPT)])


@functools.lru_cache(maxsize=None)
def _sc_kernels():
    mesh = plsc.VectorSubcoreMesh(
        core_axis_name="c", subcore_axis_name="s",
        num_cores=NC, num_subcores=NS)
    deg_k = pl.kernel(
        _deg_body,
        out_type=jax.ShapeDtypeStruct((NC * NPAD,), jnp.float32),
        mesh=mesh,
        scratch_types=[
            pltpu.VMEM((C,), jnp.float32),
            pltpu.VMEM((C,), jnp.int32),
            pltpu.VMEM_SHARED((NPAD,), jnp.float32),
        ],
    )
    hop_k = pl.kernel(
        _hop_body,
        out_type=jax.ShapeDtypeStruct((OUT, NPAD), jnp.float32),
        mesh=mesh,
        scratch_types=[
            pltpu.VMEM((FPT, NPAD), jnp.float32),
            pltpu.VMEM((FPT, NPAD), jnp.float32),
            pltpu.VMEM((CH,), jnp.int32),
            pltpu.VMEM((CH,), jnp.int32),
            pltpu.VMEM((CH,), jnp.int32),
            pltpu.VMEM((CH,), jnp.int32),
        ] + [pltpu.SemaphoreType.DMA] * 4,
        compiler_params=pltpu.CompilerParams(
            use_tc_tiling_on_sc=False, needs_layout_passes=False),
    )
    return deg_k, hop_k


_BLK = 1024
_Z = np.int32(0)


def _mlp_body(x_ref, w1t_ref, b1_ref, w2t_ref, b2_ref, o_ref):
    h = jnp.maximum(
        jnp.dot(x_ref[...], w1t_ref[...], preferred_element_type=jnp.float32)
        + b1_ref[...], 0.0)
    o_ref[...] = (
        jnp.dot(h, w2t_ref[...], preferred_element_type=jnp.float32)
        + b2_ref[...])


def _mlp(x, w1t, b1, w2t, b2):
    grid = NPAD // _BLK
    return pl.pallas_call(
        _mlp_body,
        grid=(grid,),
        in_specs=[
            pl.BlockSpec((_BLK, IN_DIM), lambda i: (i, _Z)),
            pl.BlockSpec((IN_DIM, OUT), lambda i: (_Z, _Z)),
            pl.BlockSpec((1, OUT), lambda i: (_Z, _Z)),
            pl.BlockSpec((OUT, OUT), lambda i: (_Z, _Z)),
            pl.BlockSpec((1, OUT), lambda i: (_Z, _Z)),
        ],
        out_specs=pl.BlockSpec((_BLK, OUT), lambda i: (i, _Z)),
        out_shape=jax.ShapeDtypeStruct((NPAD, OUT), jnp.float32),
    )(x, w1t, b1, w2t, b2)


def _norm_body(deg_ref, o_ref):
    d = jnp.sum(deg_ref[...], axis=1, keepdims=True)
    o_ref[...] = jnp.where(d > 0.0, lax.rsqrt(jnp.maximum(d, 1.0)), 0.0)


def _norm(degs_t):
    grid = NPAD // _BLK
    return pl.pallas_call(
        _norm_body,
        grid=(grid,),
        in_specs=[pl.BlockSpec((_BLK, NC), lambda i: (i, _Z))],
        out_specs=pl.BlockSpec((_BLK, 1), lambda i: (i, _Z)),
        out_shape=jax.ShapeDtypeStruct((NPAD, 1), jnp.float32),
    )(degs_t)


def _scale0_body(h_ref, n_ref, sw_ref, sb_ref, x0_ref, acc_ref):
    h = h_ref[...]
    x0_ref[...] = (h * n_ref[...]).T
    score = jnp.sum(h * sw_ref[...], axis=1, keepdims=True) + sb_ref[...]
    acc_ref[...] = (h * jax.nn.sigmoid(score)).T


def _scale0(h, norm, sw, sb):
    grid = NPAD // _BLK
    return pl.pallas_call(
        _scale0_body,
        grid=(grid,),
        in_specs=[
            pl.BlockSpec((_BLK, OUT), lambda i: (i, _Z)),
            pl.BlockSpec((_BLK, 1), lambda i: (i, _Z)),
            pl.BlockSpec((1, OUT), lambda i: (_Z, _Z)),
            pl.BlockSpec((1, 1), lambda i: (_Z, _Z)),
        ],
        out_specs=[
            pl.BlockSpec((OUT, _BLK), lambda i: (_Z, i)),
            pl.BlockSpec((OUT, _BLK), lambda i: (_Z, i)),
        ],
        out_shape=[
            jax.ShapeDtypeStruct((OUT, NPAD), jnp.float32),
            jax.ShapeDtypeStruct((OUT, NPAD), jnp.float32),
        ],
    )(h, norm, sw, sb)


def _glue_body(y_ref, n_ref, sw_ref, sb_ref, acc_ref, xk_ref, accout_ref):
    nrm = n_ref[...]
    feats = y_ref[...] * nrm
    xk_ref[...] = feats * nrm
    score = jnp.sum(feats * sw_ref[...], axis=0, keepdims=True) + sb_ref[...]
    accout_ref[...] = acc_ref[...] + feats * jax.nn.sigmoid(score)


def _glue(y_t, norm_t, sw_t, sb, acc_t):
    grid = NPAD // _BLK
    return pl.pallas_call(
        _glue_body,
        grid=(grid,),
        in_specs=[
            pl.BlockSpec((OUT, _BLK), lambda i: (_Z, i)),
            pl.BlockSpec((1, _BLK), lambda i: (_Z, i)),
            pl.BlockSpec((OUT, 1), lambda i: (_Z, _Z)),
            pl.BlockSpec((1, 1), lambda i: (_Z, _Z)),
            pl.BlockSpec((OUT, _BLK), lambda i: (_Z, i)),
        ],
        out_specs=[
            pl.BlockSpec((OUT, _BLK), lambda i: (_Z, i)),
            pl.BlockSpec((OUT, _BLK), lambda i: (_Z, i)),
        ],
        out_shape=[
            jax.ShapeDtypeStruct((OUT, NPAD), jnp.float32),
            jax.ShapeDtypeStruct((OUT, NPAD), jnp.float32),
        ],
    )(y_t, norm_t, sw_t, sb, acc_t)


def kernel(features, edge_index, W1, b1, W2, b2, sW, sb):
    src = edge_index[0].astype(jnp.int32)
    dst = edge_index[1].astype(jnp.int32)

    fpad = jnp.zeros((NPAD, IN_DIM), jnp.float32).at[:N].set(features)
    w1t = W1.T
    w2t = W2.T
    b1r = b1.reshape(1, OUT)
    b2r = b2.reshape(1, OUT)
    swr = sW.reshape(1, OUT)
    swt = sW.reshape(OUT, 1)
    sbr = sb.reshape(1, 1)

    zeros_c = jnp.zeros((C,), jnp.float32)
    ones_c = jnp.ones((C,), jnp.float32)
    zeros2 = jnp.zeros((FPT, NPAD), jnp.float32)

    deg_k, hop_k = _sc_kernels()
    h = _mlp(fpad, w1t, b1r, w2t, b2r)
    degs = deg_k(dst, zeros_c, ones_c)
    norm = _norm(degs.reshape(NC, NPAD).T)
    norm_t = norm.T
    x_t, acc_t = _scale0(h, norm, swr, sbr)
    for _ in range(K):
        y_t = hop_k(x_t, src, dst, zeros2)
        x_t, acc_t = _glue(y_t, norm_t, swt, sbr, acc_t)
    return acc_t[:, :N].T


# speedup vs baseline: 1.0219x; 1.0219x over previous
import functools

import numpy as np

import jax
import jax.numpy as jnp
from jax import lax
from jax.experimental import pallas as pl
from jax.experimental.pallas import tpu as pltpu
from jax.experimental.pallas import tpu_sc as plsc

N = 10000
E = 320000
IN_DIM = 128
OUT = 64
K = 12

NC = 2
NS = 16
NW = NC * NS
NPAD = 10240
RPS = NPAD // NS
EPW = E // NW
C = 80
NCHUNK = EPW // C
FPT = OUT // NW
CH = 6400
NCHK = E // CH

def _deg_body(dst_hbm, zeros_hbm, ones_hbm, out_hbm, valb, dstb, acc_sh):
    c = lax.axis_index("c")
    s = lax.axis_index("s")
    w = c * jnp.int32(NS) + s
    pltpu.sync_copy(zeros_hbm, valb)
    for j in range(RPS // C):
        pltpu.sync_copy(valb, acc_sh.at[pl.ds(s * jnp.int32(RPS) + jnp.int32(j * C), C)])
    plsc.subcore_barrier()
    pltpu.sync_copy(ones_hbm, valb)

    @pl.loop(jnp.int32(0), jnp.int32(NCHUNK))
    def _chunks(i):
        base = w * jnp.int32(EPW) + i * jnp.int32(C)
        pltpu.sync_copy(dst_hbm.at[pl.ds(base, C)], dstb)
        pltpu.sync_copy(valb, acc_sh.at[dstb], add=True)
    plsc.subcore_barrier()
    for j in range(RPS // C):
        pltpu.sync_copy(acc_sh.at[pl.ds(s * jnp.int32(RPS) + jnp.int32(j * C), C)], valb)
        pltpu.sync_copy(valb, out_hbm.at[pl.ds(c * jnp.int32(NPAD) + s * jnp.int32(RPS) + jnp.int32(j * C), C)])


def _hop_body(xt_hbm, src_hbm, dst_hbm, zeros2_hbm, out_hbm,
              xsl, acc, sb0, db0, sb1, db1, i0, i1, i2, i3):
    c = lax.axis_index("c")
    s = lax.axis_index("s")
    t2 = (c * jnp.int32(NS) + s) * jnp.int32(FPT)
    pltpu.sync_copy(xt_hbm.at[pl.ds(t2, FPT)], xsl)
    pltpu.sync_copy(zeros2_hbm, acc)

    F0 = jnp.full((16,), 0, jnp.int32)
    F1 = jnp.full((16,), 1, jnp.int32)

    def issue(ch, sbuf, dbuf, ss, sd):
        base = ch * jnp.int32(CH)
        pltpu.async_copy(src_hbm.at[pl.ds(base, CH)], sbuf, ss)
        pltpu.async_copy(dst_hbm.at[pl.ds(base, CH)], dbuf, sd)

    def drain(sbuf, dbuf, ss, sd):
        pltpu.make_async_copy(
            src_hbm.at[pl.ds(jnp.int32(0), CH)], sbuf, ss).wait()
        pltpu.make_async_copy(
            dst_hbm.at[pl.ds(jnp.int32(0), CH)], dbuf, sd).wait()

    def compute(sbuf, dbuf):
        @pl.loop(jnp.int32(0), jnp.int32(CH // 128))
        def _vecs(k):
            base = k * jnp.int32(128)
            for u in range(8):
                off = base + jnp.int32(u * 16)
                sv = sbuf[pl.ds(off, 16)]
                dv = dbuf[pl.ds(off, 16)]
                v0 = plsc.load_gather(xsl, [F0, sv])
                plsc.addupdate_scatter(acc, [F0, dv], v0)
                v1 = plsc.load_gather(xsl, [F1, sv])
                plsc.addupdate_scatter(acc, [F1, dv], v1)

    issue(jnp.int32(0), sb0, db0, i0, i1)

    @pl.loop(jnp.int32(0), jnp.int32(NCHK // 2))
    def _pairs(i):
        ch = i * jnp.int32(2)
        issue(ch + jnp.int32(1), sb1, db1, i2, i3)
        drain(sb0, db0, i0, i1)
        compute(sb0, db0)
        issue(jnp.minimum(ch + jnp.int32(2), jnp.int32(NCHK - 2)),
              sb0, db0, i0, i1)
        drain(sb1, db1, i2, i3)
        compute(sb1, db1)

    drain(sb0, db0, i0, i1)
    pltpu.sync_copy(acc, out_hbm.at[pl.ds(t2, FPT)])


@functools.lru_cache(maxsize=None)
def _sc_kernels():
    mesh = plsc.VectorSubcoreMesh(
        core_axis_name="c", subcore_axis_name="s",
        num_cores=NC, num_subcores=NS)
    deg_k = pl.kernel(
        _deg_body,
        out_type=jax.ShapeDtypeStruct((NC * NPAD,), jnp.float32),
        mesh=mesh,
        scratch_types=[
            pltpu.VMEM((C,), jnp.float32),
            pltpu.VMEM((C,), jnp.int32),
            pltpu.VMEM_SHARED((NPAD,), jnp.float32),
        ],
    )
    hop_k = pl.kernel(
        _hop_body,
        out_type=jax.ShapeDtypeStruct((OUT, NPAD), jnp.float32),
        mesh=mesh,
        scratch_types=[
            pltpu.VMEM((FPT, NPAD), jnp.float32),
            pltpu.VMEM((FPT, NPAD), jnp.float32),
            pltpu.VMEM((CH,), jnp.int32),
            pltpu.VMEM((CH,), jnp.int32),
            pltpu.VMEM((CH,), jnp.int32),
            pltpu.VMEM((CH,), jnp.int32),
        ] + [pltpu.SemaphoreType.DMA] * 4,
        compiler_params=pltpu.CompilerParams(
            use_tc_tiling_on_sc=False, needs_layout_passes=False),
    )
    return deg_k, hop_k


_BLK = 1024
_Z = np.int32(0)


def _mlp_body(x_ref, w1t_ref, b1_ref, w2t_ref, b2_ref, o_ref):
    h = jnp.maximum(
        jnp.dot(x_ref[...], w1t_ref[...], preferred_element_type=jnp.float32)
        + b1_ref[...], 0.0)
    o_ref[...] = (
        jnp.dot(h, w2t_ref[...], preferred_element_type=jnp.float32)
        + b2_ref[...])


def _mlp(x, w1t, b1, w2t, b2):
    grid = NPAD // _BLK
    return pl.pallas_call(
        _mlp_body,
        grid=(grid,),
        in_specs=[
            pl.BlockSpec((_BLK, IN_DIM), lambda i: (i, _Z)),
            pl.BlockSpec((IN_DIM, OUT), lambda i: (_Z, _Z)),
            pl.BlockSpec((1, OUT), lambda i: (_Z, _Z)),
            pl.BlockSpec((OUT, OUT), lambda i: (_Z, _Z)),
            pl.BlockSpec((1, OUT), lambda i: (_Z, _Z)),
        ],
        out_specs=pl.BlockSpec((_BLK, OUT), lambda i: (i, _Z)),
        out_shape=jax.ShapeDtypeStruct((NPAD, OUT), jnp.float32),
    )(x, w1t, b1, w2t, b2)


def _norm_body(deg_ref, o_ref):
    d = jnp.sum(deg_ref[...], axis=1, keepdims=True)
    o_ref[...] = jnp.where(d > 0.0, lax.rsqrt(jnp.maximum(d, 1.0)), 0.0)


def _norm(degs_t):
    grid = NPAD // _BLK
    return pl.pallas_call(
        _norm_body,
        grid=(grid,),
        in_specs=[pl.BlockSpec((_BLK, NC), lambda i: (i, _Z))],
        out_specs=pl.BlockSpec((_BLK, 1), lambda i: (i, _Z)),
        out_shape=jax.ShapeDtypeStruct((NPAD, 1), jnp.float32),
    )(degs_t)


def _scale0_body(h_ref, n_ref, sw_ref, sb_ref, x0_ref, acc_ref):
    h = h_ref[...]
    x0_ref[...] = (h * n_ref[...]).T
    score = jnp.sum(h * sw_ref[...], axis=1, keepdims=True) + sb_ref[...]
    acc_ref[...] = (h * jax.nn.sigmoid(score)).T


def _scale0(h, norm, sw, sb):
    grid = NPAD // _BLK
    return pl.pallas_call(
        _scale0_body,
        grid=(grid,),
        in_specs=[
            pl.BlockSpec((_BLK, OUT), lambda i: (i, _Z)),
            pl.BlockSpec((_BLK, 1), lambda i: (i, _Z)),
            pl.BlockSpec((1, OUT), lambda i: (_Z, _Z)),
            pl.BlockSpec((1, 1), lambda i: (_Z, _Z)),
        ],
        out_specs=[
            pl.BlockSpec((OUT, _BLK), lambda i: (_Z, i)),
            pl.BlockSpec((OUT, _BLK), lambda i: (_Z, i)),
        ],
        out_shape=[
            jax.ShapeDtypeStruct((OUT, NPAD), jnp.float32),
            jax.ShapeDtypeStruct((OUT, NPAD), jnp.float32),
        ],
    )(h, norm, sw, sb)


def _glue_body(y_ref, n_ref, sw_ref, sb_ref, acc_ref, xk_ref, accout_ref):
    nrm = n_ref[...]
    feats = y_ref[...] * nrm
    xk_ref[...] = feats * nrm
    score = jnp.sum(feats * sw_ref[...], axis=0, keepdims=True) + sb_ref[...]
    accout_ref[...] = acc_ref[...] + feats * jax.nn.sigmoid(score)


def _glue(y_t, norm_t, sw_t, sb, acc_t):
    grid = NPAD // _BLK
    return pl.pallas_call(
        _glue_body,
        grid=(grid,),
        in_specs=[
            pl.BlockSpec((OUT, _BLK), lambda i: (_Z, i)),
            pl.BlockSpec((1, _BLK), lambda i: (_Z, i)),
            pl.BlockSpec((OUT, 1), lambda i: (_Z, _Z)),
            pl.BlockSpec((1, 1), lambda i: (_Z, _Z)),
            pl.BlockSpec((OUT, _BLK), lambda i: (_Z, i)),
        ],
        out_specs=[
            pl.BlockSpec((OUT, _BLK), lambda i: (_Z, i)),
            pl.BlockSpec((OUT, _BLK), lambda i: (_Z, i)),
        ],
        out_shape=[
            jax.ShapeDtypeStruct((OUT, NPAD), jnp.float32),
            jax.ShapeDtypeStruct((OUT, NPAD), jnp.float32),
        ],
    )(y_t, norm_t, sw_t, sb, acc_t)


def kernel(features, edge_index, W1, b1, W2, b2, sW, sb):
    src = edge_index[0].astype(jnp.int32)
    dst = edge_index[1].astype(jnp.int32)

    fpad = jnp.zeros((NPAD, IN_DIM), jnp.float32).at[:N].set(features)
    w1t = W1.T
    w2t = W2.T
    b1r = b1.reshape(1, OUT)
    b2r = b2.reshape(1, OUT)
    swr = sW.reshape(1, OUT)
    swt = sW.reshape(OUT, 1)
    sbr = sb.reshape(1, 1)

    zeros_c = jnp.zeros((C,), jnp.float32)
    ones_c = jnp.ones((C,), jnp.float32)
    zeros2 = jnp.zeros((FPT, NPAD), jnp.float32)

    deg_k, hop_k = _sc_kernels()
    h = _mlp(fpad, w1t, b1r, w2t, b2r)
    degs = deg_k(dst, zeros_c, ones_c)
    norm = _norm(degs.reshape(NC, NPAD).T)
    norm_t = norm.T
    x_t, acc_t = _scale0(h, norm, swr, sbr)
    for _ in range(K):
        y_t = hop_k(x_t, src, dst, zeros2)
        x_t, acc_t = _glue(y_t, norm_t, swt, sbr, acc_t)
    return acc_t[:, :N].T


# speedup vs baseline: 1.3070x; 1.2791x over previous
import functools

import numpy as np

import jax
import jax.numpy as jnp
from jax import lax
from jax.experimental import pallas as pl
from jax.experimental.pallas import tpu as pltpu
from jax.experimental.pallas import tpu_sc as plsc

N = 10000
E = 320000
IN_DIM = 128
OUT = 64
K = 12

NC = 2
NS = 16
NW = NC * NS
NPAD = 10240
RPS = NPAD // NS
EPW = E // NW
C = 80
NCHUNK = EPW // C
HC = 128
HCHUNK = 80
EPWP = HC * HCHUNK
EPAD = NW * EPWP

def _deg_body(dst_hbm, zeros_hbm, ones_hbm, out_hbm, valb, dstb, acc_sh):
    c = lax.axis_index("c")
    s = lax.axis_index("s")
    w = c * jnp.int32(NS) + s
    pltpu.sync_copy(zeros_hbm, valb)
    for j in range(RPS // C):
        pltpu.sync_copy(valb, acc_sh.at[pl.ds(s * jnp.int32(RPS) + jnp.int32(j * C), C)])
    plsc.subcore_barrier()
    pltpu.sync_copy(ones_hbm, valb)

    @pl.loop(jnp.int32(0), jnp.int32(NCHUNK))
    def _chunks(i):
        base = w * jnp.int32(EPW) + i * jnp.int32(C)
        pltpu.sync_copy(dst_hbm.at[pl.ds(base, C)], dstb)
        pltpu.sync_copy(valb, acc_sh.at[dstb], add=True)
    plsc.subcore_barrier()
    for j in range(RPS // C):
        pltpu.sync_copy(acc_sh.at[pl.ds(s * jnp.int32(RPS) + jnp.int32(j * C), C)], valb)
        pltpu.sync_copy(valb, out_hbm.at[pl.ds(c * jnp.int32(NPAD) + s * jnp.int32(RPS) + jnp.int32(j * C), C)])


def _hop_body(x_hbm, src_hbm, dst_hbm, zrows_hbm, out_hbm,
              srcall, dstall, b0, b1, b2, b3, acc_sh,
              g0, g1, g2, g3, s0, s1, s2, s3):
    c = lax.axis_index("c")
    s = lax.axis_index("s")
    w = c * jnp.int32(NS) + s
    bufs = (b0, b1, b2, b3)
    gsems = (g0, g1, g2, g3)
    ssems = (s0, s1, s2, s3)

    def gather(ch, q):
        pltpu.async_copy(x_hbm.at[srcall.at[ch]], bufs[q], gsems[q])

    def wait_g(q):
        pltpu.make_async_copy(
            x_hbm.at[pl.ds(jnp.int32(0), HC)], bufs[q], gsems[q]).wait()

    def scat(ch, q):
        pltpu.async_copy(bufs[q], acc_sh.at[dstall.at[ch]], ssems[q],
                         add=True)

    def wait_s(q):
        pltpu.make_async_copy(
            bufs[q], acc_sh.at[pl.ds(jnp.int32(0), HC)], ssems[q]).wait()

    pltpu.sync_copy(src_hbm.at[w], srcall)
    pltpu.sync_copy(dst_hbm.at[w], dstall)
    pltpu.sync_copy(zrows_hbm, b0)
    for j in range(RPS // HC):
        pltpu.sync_copy(b0, acc_sh.at[pl.ds(s * jnp.int32(RPS) + jnp.int32(j * HC), HC)])
    plsc.subcore_barrier()

    for q in range(4):
        gather(jnp.int32(q), q)

    @pl.loop(jnp.int32(0), jnp.int32(HCHUNK // 4 - 1))
    def _groups(i):
        i4 = i * jnp.int32(4)
        for q in range(4):
            wait_g(q)
            pltpu.sync_copy(bufs[q], acc_sh.at[dstall.at[i4 + jnp.int32(q)]],
                            add=True)
            gather(i4 + jnp.int32(q + 4), q)

    L = jnp.int32(HCHUNK - 4)
    for q in range(4):
        wait_g(q)
        pltpu.sync_copy(bufs[q], acc_sh.at[dstall.at[L + jnp.int32(q)]],
                        add=True)

    plsc.subcore_barrier()
    for j in range(RPS // HC):
        pltpu.sync_copy(acc_sh.at[pl.ds(s * jnp.int32(RPS) + jnp.int32(j * HC), HC)], b0)
        pltpu.sync_copy(b0, out_hbm.at[pl.ds(c * jnp.int32(NPAD) + s * jnp.int32(RPS) + jnp.int32(j * HC), HC)])


@functools.lru_cache(maxsize=None)
def _sc_kernels():
    mesh = plsc.VectorSubcoreMesh(
        core_axis_name="c", subcore_axis_name="s",
        num_cores=NC, num_subcores=NS)
    deg_k = pl.kernel(
        _deg_body,
        out_type=jax.ShapeDtypeStruct((NC * NPAD,), jnp.float32),
        mesh=mesh,
        scratch_types=[
            pltpu.VMEM((C,), jnp.float32),
            pltpu.VMEM((C,), jnp.int32),
            pltpu.VMEM_SHARED((NPAD,), jnp.float32),
        ],
    )
    hop_k = pl.kernel(
        _hop_body,
        out_type=jax.ShapeDtypeStruct((NC * NPAD, OUT), jnp.float32),
        mesh=mesh,
        scratch_types=[
            pltpu.VMEM((HCHUNK, HC), jnp.int32),
            pltpu.VMEM((HCHUNK, HC), jnp.int32),
            pltpu.VMEM((HC, OUT), jnp.float32),
            pltpu.VMEM((HC, OUT), jnp.float32),
            pltpu.VMEM((HC, OUT), jnp.float32),
            pltpu.VMEM((HC, OUT), jnp.float32),
            pltpu.VMEM_SHARED((NPAD, OUT), jnp.float32),
        ] + [pltpu.SemaphoreType.DMA] * 8,
        compiler_params=pltpu.CompilerParams(use_tc_tiling_on_sc=False),
    )
    return deg_k, hop_k


_BLK = 1024
_Z = np.int32(0)


def _mlp_body(x_ref, w1t_ref, b1_ref, w2t_ref, b2_ref, o_ref):
    h = jnp.maximum(
        jnp.dot(x_ref[...], w1t_ref[...], preferred_element_type=jnp.float32)
        + b1_ref[...], 0.0)
    o_ref[...] = (
        jnp.dot(h, w2t_ref[...], preferred_element_type=jnp.float32)
        + b2_ref[...])


def _mlp(x, w1t, b1, w2t, b2):
    grid = NPAD // _BLK
    return pl.pallas_call(
        _mlp_body,
        grid=(grid,),
        in_specs=[
            pl.BlockSpec((_BLK, IN_DIM), lambda i: (i, _Z)),
            pl.BlockSpec((IN_DIM, OUT), lambda i: (_Z, _Z)),
            pl.BlockSpec((1, OUT), lambda i: (_Z, _Z)),
            pl.BlockSpec((OUT, OUT), lambda i: (_Z, _Z)),
            pl.BlockSpec((1, OUT), lambda i: (_Z, _Z)),
        ],
        out_specs=pl.BlockSpec((_BLK, OUT), lambda i: (i, _Z)),
        out_shape=jax.ShapeDtypeStruct((NPAD, OUT), jnp.float32),
    )(x, w1t, b1, w2t, b2)


def _norm_body(deg_ref, o_ref):
    d = jnp.sum(deg_ref[...], axis=1, keepdims=True)
    o_ref[...] = jnp.where(d > 0.0, lax.rsqrt(jnp.maximum(d, 1.0)), 0.0)


def _norm(degs_t):
    grid = NPAD // _BLK
    return pl.pallas_call(
        _norm_body,
        grid=(grid,),
        in_specs=[pl.BlockSpec((_BLK, NC), lambda i: (i, _Z))],
        out_specs=pl.BlockSpec((_BLK, 1), lambda i: (i, _Z)),
        out_shape=jax.ShapeDtypeStruct((NPAD, 1), jnp.float32),
    )(degs_t)


def _scale0_body(h_ref, n_ref, sw_ref, sb_ref, x0_ref, acc_ref):
    h = h_ref[...]
    x0_ref[...] = h * n_ref[...]
    score = jnp.sum(h * sw_ref[...], axis=1, keepdims=True) + sb_ref[...]
    acc_ref[...] = h * jax.nn.sigmoid(score)


def _scale0(h, norm, sw, sb):
    grid = NPAD // _BLK
    return pl.pallas_call(
        _scale0_body,
        grid=(grid,),
        in_specs=[
            pl.BlockSpec((_BLK, OUT), lambda i: (i, _Z)),
            pl.BlockSpec((_BLK, 1), lambda i: (i, _Z)),
            pl.BlockSpec((1, OUT), lambda i: (_Z, _Z)),
            pl.BlockSpec((1, 1), lambda i: (_Z, _Z)),
        ],
        out_specs=[
            pl.BlockSpec((_BLK, OUT), lambda i: (i, _Z)),
            pl.BlockSpec((_BLK, OUT), lambda i: (i, _Z)),
        ],
        out_shape=[
            jax.ShapeDtypeStruct((NPAD, OUT), jnp.float32),
            jax.ShapeDtypeStruct((NPAD, OUT), jnp.float32),
        ],
    )(h, norm, sw, sb)


def _glue_body(y_ref, n_ref, sw_ref, sb_ref, acc_ref, xk_ref, accout_ref):
    nrm = n_ref[...]
    feats = (y_ref[0] + y_ref[1]) * nrm
    xk_ref[...] = feats * nrm
    score = jnp.sum(feats * sw_ref[...], axis=1, keepdims=True) + sb_ref[...]
    accout_ref[...] = acc_ref[...] + feats * jax.nn.sigmoid(score)


def _glue(y, norm, sw, sb, acc):
    grid = NPAD // _BLK
    return pl.pallas_call(
        _glue_body,
        grid=(grid,),
        in_specs=[
            pl.BlockSpec((NC, _BLK, OUT), lambda i: (_Z, i, _Z)),
            pl.BlockSpec((_BLK, 1), lambda i: (i, _Z)),
            pl.BlockSpec((1, OUT), lambda i: (_Z, _Z)),
            pl.BlockSpec((1, 1), lambda i: (_Z, _Z)),
            pl.BlockSpec((_BLK, OUT), lambda i: (i, _Z)),
        ],
        out_specs=[
            pl.BlockSpec((_BLK, OUT), lambda i: (i, _Z)),
            pl.BlockSpec((_BLK, OUT), lambda i: (i, _Z)),
        ],
        out_shape=[
            jax.ShapeDtypeStruct((NPAD, OUT), jnp.float32),
            jax.ShapeDtypeStruct((NPAD, OUT), jnp.float32),
        ],
    )(y, norm, sw, sb, acc)


def kernel(features, edge_index, W1, b1, W2, b2, sW, sb):
    src = edge_index[0].astype(jnp.int32)
    dst = edge_index[1].astype(jnp.int32)

    fpad = jnp.zeros((NPAD, IN_DIM), jnp.float32).at[:N].set(features)
    w1t = W1.T
    w2t = W2.T
    b1r = b1.reshape(1, OUT)
    b2r = b2.reshape(1, OUT)
    swr = sW.reshape(1, OUT)
    sbr = sb.reshape(1, 1)

    zeros_c = jnp.zeros((C,), jnp.float32)
    ones_c = jnp.ones((C,), jnp.float32)
    zrows = jnp.zeros((HC, OUT), jnp.float32)

    src3 = jnp.zeros((EPAD,), jnp.int32).at[:E].set(src).reshape(NW, HCHUNK, HC)
    dst3 = jnp.full((EPAD,), NPAD - 1, jnp.int32).at[:E].set(dst).reshape(
        NW, HCHUNK, HC)

    deg_k, hop_k = _sc_kernels()
    h = _mlp(fpad, w1t, b1r, w2t, b2r)
    degs = deg_k(dst, zeros_c, ones_c)
    norm = _norm(degs.reshape(NC, NPAD).T)
    x, acc = _scale0(h, norm, swr, sbr)
    for _ in range(K):
        y = hop_k(x, src3, dst3, zrows)
        x, acc = _glue(y.reshape(NC, NPAD, OUT), norm, swr, sbr, acc)
    return acc[:N]


# speedup vs baseline: 1.7848x; 1.3655x over previous
import functools

import numpy as np

import jax
import jax.numpy as jnp
from jax import lax
from jax.experimental import pallas as pl
from jax.experimental.pallas import tpu as pltpu
from jax.experimental.pallas import tpu_sc as plsc

N = 10000
E = 320000
IN_DIM = 128
OUT = 64
K = 12

NC = 2
NS = 16
NW = NC * NS
NPAD = 10240
RPS = NPAD // NS
EPW = E // NW
C = 80
NCHUNK = EPW // C
HC = 128
HCHUNK = 50
EA = NW * HC * HCHUNK
FPT = OUT // NW
CB = 2304
NB = 50

def _deg_body(dst_hbm, zeros_hbm, ones_hbm, out_hbm, valb, dstb, acc_sh):
    c = lax.axis_index("c")
    s = lax.axis_index("s")
    w = c * jnp.int32(NS) + s
    pltpu.sync_copy(zeros_hbm, valb)
    for j in range(RPS // C):
        pltpu.sync_copy(valb, acc_sh.at[pl.ds(s * jnp.int32(RPS) + jnp.int32(j * C), C)])
    plsc.subcore_barrier()
    pltpu.sync_copy(ones_hbm, valb)

    @pl.loop(jnp.int32(0), jnp.int32(NCHUNK))
    def _chunks(i):
        base = w * jnp.int32(EPW) + i * jnp.int32(C)
        pltpu.sync_copy(dst_hbm.at[pl.ds(base, C)], dstb)
        pltpu.sync_copy(valb, acc_sh.at[dstb], add=True)
    plsc.subcore_barrier()
    for j in range(RPS // C):
        pltpu.sync_copy(acc_sh.at[pl.ds(s * jnp.int32(RPS) + jnp.int32(j * C), C)], valb)
        pltpu.sync_copy(valb, out_hbm.at[pl.ds(c * jnp.int32(NPAD) + s * jnp.int32(RPS) + jnp.int32(j * C), C)])


def _hop_body(x_hbm, xt_hbm, srcA_hbm, dstA_hbm, srcf_hbm, dstf_hbm,
              zrows_hbm, zeros2_hbm, out_hbm, out2_hbm,
              srcall, dstall, r0, r1, xsl, accf, sb0, db0, sb1, db1, acc_sh,
              g0, g1, i0, i1, i2, i3):
    c = lax.axis_index("c")
    s = lax.axis_index("s")
    w = c * jnp.int32(NS) + s
    t2 = w * jnp.int32(FPT)
    rbufs = (r0, r1)
    gsems = (g0, g1)

    F0 = jnp.full((16,), 0, jnp.int32)
    F1 = jnp.full((16,), 1, jnp.int32)

    def gatherA(ch, q):
        pltpu.async_copy(x_hbm.at[srcall.at[ch]], rbufs[q], gsems[q])

    def waitA(q):
        pltpu.make_async_copy(
            x_hbm.at[pl.ds(jnp.int32(0), HC)], rbufs[q], gsems[q]).wait()

    def scatA(ch, q):
        pltpu.sync_copy(rbufs[q], acc_sh.at[dstall.at[ch]], add=True)

    bbufs = ((sb0, db0, i0, i1), (sb1, db1, i2, i3))

    def issueB(ch, q):
        sbuf, dbuf, ss, sd = bbufs[q]
        base = jnp.int32(EA) + ch * jnp.int32(CB)
        pltpu.async_copy(srcf_hbm.at[pl.ds(base, CB)], sbuf, ss)
        pltpu.async_copy(dstf_hbm.at[pl.ds(base, CB)], dbuf, sd)

    def drainB(q):
        sbuf, dbuf, ss, sd = bbufs[q]
        pltpu.make_async_copy(
            srcf_hbm.at[pl.ds(jnp.int32(0), CB)], sbuf, ss).wait()
        pltpu.make_async_copy(
            dstf_hbm.at[pl.ds(jnp.int32(0), CB)], dbuf, sd).wait()

    def computeB(q):
        sbuf, dbuf = bbufs[q][0], bbufs[q][1]

        @pl.loop(jnp.int32(0), jnp.int32(CB // 64))
        def _vecs(k):
            kb = k * jnp.int32(64)
            for u in range(4):
                off = kb + jnp.int32(u * 16)
                sv = sbuf[pl.ds(off, 16)]
                dv = dbuf[pl.ds(off, 16)]
                v0 = plsc.load_gather(xsl, [F0, sv])
                plsc.addupdate_scatter(accf, [F0, dv], v0)
                v1 = plsc.load_gather(xsl, [F1, sv])
                plsc.addupdate_scatter(accf, [F1, dv], v1)

    pltpu.sync_copy(srcA_hbm.at[w], srcall)
    pltpu.sync_copy(dstA_hbm.at[w], dstall)
    pltpu.sync_copy(xt_hbm.at[pl.ds(t2, FPT)], xsl)
    pltpu.sync_copy(zeros2_hbm, accf)
    pltpu.sync_copy(zrows_hbm, r0)
    for j in range(RPS // HC):
        pltpu.sync_copy(r0, acc_sh.at[pl.ds(s * jnp.int32(RPS) + jnp.int32(j * HC), HC)])
    plsc.subcore_barrier()

    gatherA(jnp.int32(0), 0)
    issueB(jnp.int32(0), 0)

    @pl.loop(jnp.int32(0), jnp.int32(HCHUNK // 2))
    def _pairs(i):
        ch = i * jnp.int32(2)
        gatherA(ch + jnp.int32(1), 1)
        issueB(ch + jnp.int32(1), 1)
        waitA(0)
        scatA(ch, 0)
        drainB(0)
        computeB(0)
        nxt = jnp.minimum(ch + jnp.int32(2), jnp.int32(HCHUNK - 2))
        gatherA(nxt, 0)
        issueB(nxt, 0)
        waitA(1)
        scatA(ch + jnp.int32(1), 1)
        drainB(1)
        computeB(1)

    waitA(0)
    drainB(0)

    plsc.subcore_barrier()
    for j in range(RPS // HC):
        pltpu.sync_copy(acc_sh.at[pl.ds(s * jnp.int32(RPS) + jnp.int32(j * HC), HC)], r0)
        pltpu.sync_copy(r0, out_hbm.at[pl.ds(c * jnp.int32(NPAD) + s * jnp.int32(RPS) + jnp.int32(j * HC), HC)])
    pltpu.sync_copy(accf, out2_hbm.at[pl.ds(t2, FPT)])


@functools.lru_cache(maxsize=None)
def _sc_kernels():
    mesh = plsc.VectorSubcoreMesh(
        core_axis_name="c", subcore_axis_name="s",
        num_cores=NC, num_subcores=NS)
    deg_k = pl.kernel(
        _deg_body,
        out_type=jax.ShapeDtypeStruct((NC * NPAD,), jnp.float32),
        mesh=mesh,
        scratch_types=[
            pltpu.VMEM((C,), jnp.float32),
            pltpu.VMEM((C,), jnp.int32),
            pltpu.VMEM_SHARED((NPAD,), jnp.float32),
        ],
    )
    hop_k = pl.kernel(
        _hop_body,
        out_type=(jax.ShapeDtypeStruct((NC * NPAD, OUT), jnp.float32),
                  jax.ShapeDtypeStruct((OUT, NPAD), jnp.float32)),
        mesh=mesh,
        scratch_types=[
            pltpu.VMEM((HCHUNK, HC), jnp.int32),
            pltpu.VMEM((HCHUNK, HC), jnp.int32),
            pltpu.VMEM((HC, OUT), jnp.float32),
            pltpu.VMEM((HC, OUT), jnp.float32),
            pltpu.VMEM((FPT, NPAD), jnp.float32),
            pltpu.VMEM((FPT, NPAD), jnp.float32),
            pltpu.VMEM((CB,), jnp.int32),
            pltpu.VMEM((CB,), jnp.int32),
            pltpu.VMEM((CB,), jnp.int32),
            pltpu.VMEM((CB,), jnp.int32),
            pltpu.VMEM_SHARED((NPAD, OUT), jnp.float32),
        ] + [pltpu.SemaphoreType.DMA] * 6,
        compiler_params=pltpu.CompilerParams(
            use_tc_tiling_on_sc=False, needs_layout_passes=False),
    )
    return deg_k, hop_k


_BLK = 1024
_Z = np.int32(0)


def _mlp_body(x_ref, w1t_ref, b1_ref, w2t_ref, b2_ref, o_ref):
    h = jnp.maximum(
        jnp.dot(x_ref[...], w1t_ref[...], preferred_element_type=jnp.float32)
        + b1_ref[...], 0.0)
    o_ref[...] = (
        jnp.dot(h, w2t_ref[...], preferred_element_type=jnp.float32)
        + b2_ref[...])


def _mlp(x, w1t, b1, w2t, b2):
    grid = NPAD // _BLK
    return pl.pallas_call(
        _mlp_body,
        grid=(grid,),
        in_specs=[
            pl.BlockSpec((_BLK, IN_DIM), lambda i: (i, _Z)),
            pl.BlockSpec((IN_DIM, OUT), lambda i: (_Z, _Z)),
            pl.BlockSpec((1, OUT), lambda i: (_Z, _Z)),
            pl.BlockSpec((OUT, OUT), lambda i: (_Z, _Z)),
            pl.BlockSpec((1, OUT), lambda i: (_Z, _Z)),
        ],
        out_specs=pl.BlockSpec((_BLK, OUT), lambda i: (i, _Z)),
        out_shape=jax.ShapeDtypeStruct((NPAD, OUT), jnp.float32),
    )(x, w1t, b1, w2t, b2)


def _norm_body(deg_ref, o_ref):
    d = jnp.sum(deg_ref[...], axis=1, keepdims=True)
    o_ref[...] = jnp.where(d > 0.0, lax.rsqrt(jnp.maximum(d, 1.0)), 0.0)


def _norm(degs_t):
    grid = NPAD // _BLK
    return pl.pallas_call(
        _norm_body,
        grid=(grid,),
        in_specs=[pl.BlockSpec((_BLK, NC), lambda i: (i, _Z))],
        out_specs=pl.BlockSpec((_BLK, 1), lambda i: (i, _Z)),
        out_shape=jax.ShapeDtypeStruct((NPAD, 1), jnp.float32),
    )(degs_t)


def _scale0_body(h_ref, n_ref, sw_ref, sb_ref, x0_ref, x0t_ref, acc_ref):
    h = h_ref[...]
    x0 = h * n_ref[...]
    x0_ref[...] = x0
    x0t_ref[...] = x0.T
    score = jnp.sum(h * sw_ref[...], axis=1, keepdims=True) + sb_ref[...]
    acc_ref[...] = h * jax.nn.sigmoid(score)


def _scale0(h, norm, sw, sb):
    grid = NPAD // _BLK
    return pl.pallas_call(
        _scale0_body,
        grid=(grid,),
        in_specs=[
            pl.BlockSpec((_BLK, OUT), lambda i: (i, _Z)),
            pl.BlockSpec((_BLK, 1), lambda i: (i, _Z)),
            pl.BlockSpec((1, OUT), lambda i: (_Z, _Z)),
            pl.BlockSpec((1, 1), lambda i: (_Z, _Z)),
        ],
        out_specs=[
            pl.BlockSpec((_BLK, OUT), lambda i: (i, _Z)),
            pl.BlockSpec((OUT, _BLK), lambda i: (_Z, i)),
            pl.BlockSpec((_BLK, OUT), lambda i: (i, _Z)),
        ],
        out_shape=[
            jax.ShapeDtypeStruct((NPAD, OUT), jnp.float32),
            jax.ShapeDtypeStruct((OUT, NPAD), jnp.float32),
            jax.ShapeDtypeStruct((NPAD, OUT), jnp.float32),
        ],
    )(h, norm, sw, sb)


def _glue_body(y_ref, y2t_ref, n_ref, sw_ref, sb_ref, acc_ref,
               xk_ref, xkt_ref, accout_ref):
    nrm = n_ref[...]
    feats = (y_ref[0] + y_ref[1] + y2t_ref[...].T) * nrm
    xk = feats * nrm
    xk_ref[...] = xk
    xkt_ref[...] = xk.T
    score = jnp.sum(feats * sw_ref[...], axis=1, keepdims=True) + sb_ref[...]
    accout_ref[...] = acc_ref[...] + feats * jax.nn.sigmoid(score)


def _glue(y, y2t, norm, sw, sb, acc):
    grid = NPAD // _BLK
    return pl.pallas_call(
        _glue_body,
        grid=(grid,),
        in_specs=[
            pl.BlockSpec((NC, _BLK, OUT), lambda i: (_Z, i, _Z)),
            pl.BlockSpec((OUT, _BLK), lambda i: (_Z, i)),
            pl.BlockSpec((_BLK, 1), lambda i: (i, _Z)),
            pl.BlockSpec((1, OUT), lambda i: (_Z, _Z)),
            pl.BlockSpec((1, 1), lambda i: (_Z, _Z)),
            pl.BlockSpec((_BLK, OUT), lambda i: (i, _Z)),
        ],
        out_specs=[
            pl.BlockSpec((_BLK, OUT), lambda i: (i, _Z)),
            pl.BlockSpec((OUT, _BLK), lambda i: (_Z, i)),
            pl.BlockSpec((_BLK, OUT), lambda i: (i, _Z)),
        ],
        out_shape=[
            jax.ShapeDtypeStruct((NPAD, OUT), jnp.float32),
            jax.ShapeDtypeStruct((OUT, NPAD), jnp.float32),
            jax.ShapeDtypeStruct((NPAD, OUT), jnp.float32),
        ],
    )(y, y2t, norm, sw, sb, acc)


def kernel(features, edge_index, W1, b1, W2, b2, sW, sb):
    src = edge_index[0].astype(jnp.int32)
    dst = edge_index[1].astype(jnp.int32)

    fpad = jnp.zeros((NPAD, IN_DIM), jnp.float32).at[:N].set(features)
    w1t = W1.T
    w2t = W2.T
    b1r = b1.reshape(1, OUT)
    b2r = b2.reshape(1, OUT)
    swr = sW.reshape(1, OUT)
    sbr = sb.reshape(1, 1)

    zeros_c = jnp.zeros((C,), jnp.float32)
    ones_c = jnp.ones((C,), jnp.float32)
    zrows = jnp.zeros((HC, OUT), jnp.float32)
    zeros2 = jnp.zeros((FPT, NPAD), jnp.float32)

    srcA3 = src[:EA].reshape(NW, HCHUNK, HC)
    dstA3 = dst[:EA].reshape(NW, HCHUNK, HC)

    deg_k, hop_k = _sc_kernels()
    h = _mlp(fpad, w1t, b1r, w2t, b2r)
    degs = deg_k(dst, zeros_c, ones_c)
    norm = _norm(degs.reshape(NC, NPAD).T)
    x, x_t, acc = _scale0(h, norm, swr, sbr)
    for _ in range(K):
        y, y2t = hop_k(x, x_t, srcA3, dstA3, src, dst, zrows, zeros2)
        x, x_t, acc = _glue(y.reshape(NC, NPAD, OUT), y2t, norm, swr, sbr, acc)
    return acc[:N]
